# baseline (speedup 1.0000x reference)
import functools

import jax
import jax.numpy as jnp
from jax import lax
from jax.experimental import pallas as pl
from jax.experimental.pallas import tpu as pltpu
from jax.experimental.pallas import tpu_sc as plsc

N = 10000
D = 128
E = 320000
NC = 2
NS = 16
L = 16
NW = NC * NS
B = 80
CHUNKS = E // NW // B
NSEG = 5
SEGC = CHUNKS // NSEG
NR = 10240
RPT = NR // NS
NP = NR
DPT = NP // NS

_mesh = plsc.VectorSubcoreMesh(core_axis_name="c", subcore_axis_name="s")


@functools.partial(
    pl.kernel,
    out_type=jax.ShapeDtypeStruct((NC, NP), jnp.float32),
    mesh=_mesh,
    scratch_types=[
        pltpu.VMEM((NSEG, SEGC, B), jnp.int32),
        pltpu.VMEM((B,), jnp.float32),
        pltpu.VMEM((DPT,), jnp.float32),
        pltpu.VMEM_SHARED((NP,), jnp.float32),
        pltpu.SemaphoreType.DMA,
    ],
)
def _deg(dst_hbm, deg_hbm, dstv, ones, degv, acc, sem):
    c = lax.axis_index("c")
    s = lax.axis_index("s")
    w = c * NS + s
    pltpu.sync_copy(dst_hbm.at[w], dstv)

    def fill(i, _):
        ones[pl.ds(i * L, L)] = jnp.full((L,), 1.0, jnp.float32)
        return 0

    lax.fori_loop(0, B // L, fill, 0)

    def zero(i, _):
        degv[pl.ds(i * L, L)] = jnp.zeros((L,), jnp.float32)
        return 0

    lax.fori_loop(0, DPT // L, zero, 0)
    pltpu.sync_copy(degv, acc.at[pl.ds(s * DPT, DPT)])
    plsc.subcore_barrier()

    def drain(jl, _):
        pltpu.make_async_copy(deg_hbm.at[0, pl.ds(0, B)], ones, sem).wait()
        return 0

    for o in range(NSEG):

        def fire(jl, _):
            pltpu.async_copy(ones, acc.at[dstv.at[o, jl]], sem, add=True)
            return 0

        lax.fori_loop(0, SEGC, fire, 0)
        if o >= 1:
            lax.fori_loop(0, SEGC, drain, 0)
    lax.fori_loop(0, SEGC, drain, 0)
    plsc.subcore_barrier()
    pltpu.sync_copy(acc.at[pl.ds(s * DPT, DPT)], degv)
    pltpu.sync_copy(degv, deg_hbm.at[c, pl.ds(s * DPT, DPT)])


@functools.partial(
    pl.kernel,
    out_type=jax.ShapeDtypeStruct((NC, NR, D), jnp.float32),
    mesh=_mesh,
    scratch_types=[
        pltpu.VMEM((SEGC, B), jnp.int32),
        pltpu.VMEM((SEGC, B), jnp.int32),
        pltpu.VMEM((SEGC, B), jnp.int32),
        pltpu.VMEM((SEGC, B), jnp.int32),
        pltpu.VMEM((B, D), jnp.float32),
        pltpu.VMEM((B, D), jnp.float32),
        pltpu.VMEM((B, D), jnp.float32),
        pltpu.VMEM_SHARED((NR, D), jnp.float32),
    ]
    + [pltpu.SemaphoreType.DMA] * 8,
)
def _agg(hp_hbm, src_hbm, dst_hbm, zeros_hbm, out_hbm,
         srcv0, srcv1, dstv0, dstv1, buf0, buf1, buf2, acc,
         gsem0, gsem1, gsem2, ssem0, ssem1, ssem2, isem0, isem1):
    srcv = (srcv0, srcv1)
    dstv = (dstv0, dstv1)
    bufs = (buf0, buf1, buf2)
    gsem = (gsem0, gsem1, gsem2)
    ssem = (ssem0, ssem1, ssem2)
    isem = (isem0, isem1)
    c = lax.axis_index("c")
    s = lax.axis_index("s")
    w = c * NS + s
    r0 = s * RPT

    @pl.when(c == 0)
    def _():
        pltpu.sync_copy(hp_hbm.at[pl.ds(r0, RPT)], acc.at[pl.ds(r0, RPT)])

    @pl.when(c != 0)
    def _():
        pltpu.sync_copy(zeros_hbm, acc.at[pl.ds(r0, RPT)])

    pltpu.async_copy(src_hbm.at[w, 0], srcv[0], isem[0])
    pltpu.async_copy(dst_hbm.at[w, 0], dstv[0], isem[0])
    plsc.subcore_barrier()

    for o in range(NSEG):
        sl = o % 2
        nsl = (o + 1) % 2
        pltpu.make_async_copy(src_hbm.at[w, o], srcv[sl], isem[sl]).wait()
        pltpu.make_async_copy(src_hbm.at[w, o], dstv[sl], isem[sl]).wait()
        if o + 1 < NSEG:
            pltpu.async_copy(src_hbm.at[w, o + 1], srcv[nsl], isem[nsl])
            pltpu.async_copy(dst_hbm.at[w, o + 1], dstv[nsl], isem[nsl])
        sv, dv = srcv[sl], dstv[sl]
        pltpu.async_copy(hp_hbm.at[sv.at[0]], bufs[0], gsem[0])
        pltpu.async_copy(hp_hbm.at[sv.at[1]], bufs[1], gsem[1])

        def turn(jl, b):
            pb = (b - 1) % 3
            pltpu.make_async_copy(
                hp_hbm.at[pl.ds(0, B)], bufs[b], gsem[b]).wait()
            pltpu.async_copy(bufs[b], acc.at[dv.at[jl]], ssem[b], add=True)
            jn = jl + 2

            @pl.when(jn < SEGC)
            def _():
                @pl.when(jl > 0)
                def _():
                    pltpu.make_async_copy(
                        hp_hbm.at[pl.ds(0, B)], bufs[pb], ssem[pb]).wait()

                pltpu.async_copy(hp_hbm.at[sv.at[jn]], bufs[pb], gsem[pb])

        def tri(k, _):
            for b in range(3):
                turn(k * 3 + b, b)
            return 0

        lax.fori_loop(0, SEGC // 3, tri, 0)
        turn(SEGC - 1, (SEGC - 1) % 3)
        for b in range(3):
            pltpu.make_async_copy(
                hp_hbm.at[pl.ds(0, B)], bufs[b], ssem[b]).wait()

    plsc.subcore_barrier()
    pltpu.sync_copy(acc.at[pl.ds(r0, RPT)], out_hbm.at[c, pl.ds(r0, RPT)])


BN = 1024
GRID = NR // BN


def _prep_body(d0_ref, d1_ref, x_ref, w_ref, out_ref, dinv_ref):
    dinv = lax.rsqrt(d0_ref[...] + d1_ref[...] + 1.0)
    h = jnp.dot(x_ref[...], w_ref[...], preferred_element_type=jnp.float32)
    out_ref[...] = h * dinv
    dinv_ref[...] = dinv


BNP = NR // NS


_prep = pl.pallas_call(
    _prep_body,
    grid=(NS,),
    in_specs=[
        pl.BlockSpec((BNP, 1), lambda i: (i, 0)),
        pl.BlockSpec((BNP, 1), lambda i: (i, 0)),
        pl.BlockSpec((BNP, D), lambda i: (i, 0)),
        pl.BlockSpec((D, D), lambda i: (0, 0)),
    ],
    out_specs=[
        pl.BlockSpec((BNP, D), lambda i: (i, 0)),
        pl.BlockSpec((BNP, 1), lambda i: (i, 0)),
    ],
    out_shape=[
        jax.ShapeDtypeStruct((NR, D), jnp.float32),
        jax.ShapeDtypeStruct((NR, 1), jnp.float32),
    ],
)


def _mid_body(agg_ref, dinv_ref, b_ref, w_ref, out_ref):
    a = agg_ref[0] + agg_ref[1]
    y = jnp.maximum(a * dinv_ref[...] + b_ref[...], 0.0)
    out_ref[...] = (
        jnp.dot(y, w_ref[...], preferred_element_type=jnp.float32) * dinv_ref[...]
    )


_mid = pl.pallas_call(
    _mid_body,
    grid=(GRID,),
    in_specs=[
        pl.BlockSpec((NC, BN, D), lambda i: (0, i, 0)),
        pl.BlockSpec((BN, 1), lambda i: (i, 0)),
        pl.BlockSpec((1, D), lambda i: (0, 0)),
        pl.BlockSpec((D, D), lambda i: (0, 0)),
    ],
    out_specs=pl.BlockSpec((BN, D), lambda i: (i, 0)),
    out_shape=jax.ShapeDtypeStruct((NR, D), jnp.float32),
)


def _fin_body(agg_ref, dinv_ref, b_ref, out_ref):
    a = agg_ref[0] + agg_ref[1]
    out_ref[...] = a * dinv_ref[...] + b_ref[...]


BNF = 1000


_fin = pl.pallas_call(
    _fin_body,
    grid=(N // BNF,),
    in_specs=[
        pl.BlockSpec((NC, BNF, D), lambda i: (0, i, 0)),
        pl.BlockSpec((BNF, 1), lambda i: (i, 0)),
        pl.BlockSpec((1, D), lambda i: (0, 0)),
    ],
    out_specs=pl.BlockSpec((BNF, D), lambda i: (i, 0)),
    out_shape=jax.ShapeDtypeStruct((N, D), jnp.float32),
)


def kernel(x, edge_index, W1, b1, W2, b2):
    ei = edge_index.astype(jnp.int32)
    src = ei[0].reshape(NW, NSEG, SEGC, B)
    dst = ei[1].reshape(NW, NSEG, SEGC, B)

    deg2 = _deg(dst)
    zeros = jnp.zeros((RPT, D), jnp.float32)

    h1p, dinv = _prep(deg2[0].reshape(NP, 1), deg2[1].reshape(NP, 1), x, W1)
    agg1 = _agg(h1p, src, dst, zeros)
    h2p = _mid(agg1, dinv, b1.reshape(1, D), W2)
    agg2 = _agg(h2p, src, dst, zeros)
    return _fin(agg2, dinv, b2.reshape(1, D))


# speedup vs baseline: 1.0012x; 1.0012x over previous
import functools

import jax
import jax.numpy as jnp
from jax import lax
from jax.experimental import pallas as pl
from jax.experimental.pallas import tpu as pltpu
from jax.experimental.pallas import tpu_sc as plsc

N = 10000
D = 128
E = 320000
NC = 2
NS = 16
L = 16
NW = NC * NS
B = 80
CHUNKS = E // NW // B
NSEG = 5
SEGC = CHUNKS // NSEG
NR = 10240
RPT = NR // NS
NP = NR
DPT = NP // NS

_mesh = plsc.VectorSubcoreMesh(core_axis_name="c", subcore_axis_name="s")


@functools.partial(
    pl.kernel,
    out_type=jax.ShapeDtypeStruct((NC, NP), jnp.float32),
    mesh=_mesh,
    scratch_types=[
        pltpu.VMEM((NSEG, SEGC, B), jnp.int32),
        pltpu.VMEM((B,), jnp.float32),
        pltpu.VMEM((DPT,), jnp.float32),
        pltpu.VMEM_SHARED((NP,), jnp.float32),
        pltpu.SemaphoreType.DMA,
    ],
)
def _deg(dst_hbm, deg_hbm, dstv, ones, degv, acc, sem):
    c = lax.axis_index("c")
    s = lax.axis_index("s")
    w = c * NS + s
    pltpu.sync_copy(dst_hbm.at[w], dstv)

    def fill(i, _):
        ones[pl.ds(i * L, L)] = jnp.full((L,), 1.0, jnp.float32)
        return 0

    lax.fori_loop(0, B // L, fill, 0)

    def zero(i, _):
        degv[pl.ds(i * L, L)] = jnp.zeros((L,), jnp.float32)
        return 0

    lax.fori_loop(0, DPT // L, zero, 0)
    pltpu.sync_copy(degv, acc.at[pl.ds(s * DPT, DPT)])
    plsc.subcore_barrier()

    def drain(jl, _):
        pltpu.make_async_copy(deg_hbm.at[0, pl.ds(0, B)], ones, sem).wait()
        return 0

    for o in range(NSEG):

        def fire(jl, _):
            pltpu.async_copy(ones, acc.at[dstv.at[o, jl]], sem, add=True)
            return 0

        lax.fori_loop(0, SEGC, fire, 0)
        if o >= 1:
            lax.fori_loop(0, SEGC, drain, 0)
    lax.fori_loop(0, SEGC, drain, 0)
    plsc.subcore_barrier()
    pltpu.sync_copy(acc.at[pl.ds(s * DPT, DPT)], degv)
    pltpu.sync_copy(degv, deg_hbm.at[c, pl.ds(s * DPT, DPT)])


@functools.partial(
    pl.kernel,
    out_type=jax.ShapeDtypeStruct((NC, NR, D), jnp.float32),
    mesh=_mesh,
    scratch_types=[
        pltpu.VMEM((SEGC, B), jnp.int32),
        pltpu.VMEM((SEGC, B), jnp.int32),
        pltpu.VMEM((SEGC, B), jnp.int32),
        pltpu.VMEM((SEGC, B), jnp.int32),
        pltpu.VMEM((B, D), jnp.float32),
        pltpu.VMEM((B, D), jnp.float32),
        pltpu.VMEM((B, D), jnp.float32),
        pltpu.VMEM_SHARED((NR, D), jnp.float32),
    ]
    + [pltpu.SemaphoreType.DMA] * 8,
)
def _agg(hp_hbm, src_hbm, dst_hbm, zeros_hbm, out_hbm,
         srcv0, srcv1, dstv0, dstv1, buf0, buf1, buf2, acc,
         gsem0, gsem1, gsem2, ssem0, ssem1, ssem2, isem0, isem1):
    srcv = (srcv0, srcv1)
    dstv = (dstv0, dstv1)
    bufs = (buf0, buf1, buf2)
    gsem = (gsem0, gsem1, gsem2)
    ssem = (ssem0, ssem1, ssem2)
    isem = (isem0, isem1)
    c = lax.axis_index("c")
    s = lax.axis_index("s")
    w = c * NS + s
    r0 = s * RPT

    @pl.when(c == 0)
    def _():
        pltpu.sync_copy(hp_hbm.at[pl.ds(r0, RPT)], acc.at[pl.ds(r0, RPT)])

    @pl.when(c != 0)
    def _():
        pltpu.sync_copy(zeros_hbm, acc.at[pl.ds(r0, RPT)])

    pltpu.async_copy(src_hbm.at[w, 0], srcv[0], isem[0])
    pltpu.async_copy(dst_hbm.at[w, 0], dstv[0], isem[0])
    plsc.subcore_barrier()

    for o in range(NSEG):
        sl = o % 2
        nsl = (o + 1) % 2
        pltpu.make_async_copy(src_hbm.at[w, o], srcv[sl], isem[sl]).wait()
        pltpu.make_async_copy(src_hbm.at[w, o], dstv[sl], isem[sl]).wait()
        if o + 1 < NSEG:
            pltpu.async_copy(src_hbm.at[w, o + 1], srcv[nsl], isem[nsl])
            pltpu.async_copy(dst_hbm.at[w, o + 1], dstv[nsl], isem[nsl])
        sv, dv = srcv[sl], dstv[sl]
        pltpu.async_copy(hp_hbm.at[sv.at[0]], bufs[0], gsem[0])
        pltpu.async_copy(hp_hbm.at[sv.at[1]], bufs[1], gsem[1])

        def turn(jl, b):
            pb = (b - 1) % 3
            pltpu.make_async_copy(
                hp_hbm.at[pl.ds(0, B)], bufs[b], gsem[b]).wait()
            pltpu.async_copy(bufs[b], acc.at[dv.at[jl]], ssem[b], add=True)
            jn = jl + 2

            @pl.when(jn < SEGC)
            def _():
                @pl.when(jl > 0)
                def _():
                    pltpu.make_async_copy(
                        hp_hbm.at[pl.ds(0, B)], bufs[pb], ssem[pb]).wait()

                pltpu.async_copy(hp_hbm.at[sv.at[jn]], bufs[pb], gsem[pb])

        def tri(k, _):
            for b in range(3):
                turn(k * 3 + b, b)
            return 0

        lax.fori_loop(0, SEGC // 3, tri, 0)
        turn(SEGC - 1, (SEGC - 1) % 3)
        for b in range(3):
            pltpu.make_async_copy(
                hp_hbm.at[pl.ds(0, B)], bufs[b], ssem[b]).wait()

    plsc.subcore_barrier()
    pltpu.sync_copy(acc.at[pl.ds(r0, RPT)], out_hbm.at[c, pl.ds(r0, RPT)])


BN = 1024
GRID = NR // BN


def _prep_body(d0_ref, d1_ref, x_ref, w_ref, out_ref, dinv_ref):
    dinv = lax.rsqrt(d0_ref[...] + d1_ref[...] + 1.0)
    h = jnp.dot(x_ref[...], w_ref[...], preferred_element_type=jnp.float32)
    out_ref[...] = h * dinv
    dinv_ref[...] = dinv


_prep = pl.pallas_call(
    _prep_body,
    grid=(GRID,),
    in_specs=[
        pl.BlockSpec((BN, 1), lambda i: (i, 0)),
        pl.BlockSpec((BN, 1), lambda i: (i, 0)),
        pl.BlockSpec((BN, D), lambda i: (i, 0)),
        pl.BlockSpec((D, D), lambda i: (0, 0)),
    ],
    out_specs=[
        pl.BlockSpec((BN, D), lambda i: (i, 0)),
        pl.BlockSpec((BN, 1), lambda i: (i, 0)),
    ],
    out_shape=[
        jax.ShapeDtypeStruct((NR, D), jnp.float32),
        jax.ShapeDtypeStruct((NR, 1), jnp.float32),
    ],
)


def _mid_body(agg_ref, dinv_ref, b_ref, w_ref, out_ref):
    a = agg_ref[0] + agg_ref[1]
    y = jnp.maximum(a * dinv_ref[...] + b_ref[...], 0.0)
    out_ref[...] = (
        jnp.dot(y, w_ref[...], preferred_element_type=jnp.float32) * dinv_ref[...]
    )


_mid = pl.pallas_call(
    _mid_body,
    grid=(GRID,),
    in_specs=[
        pl.BlockSpec((NC, BN, D), lambda i: (0, i, 0)),
        pl.BlockSpec((BN, 1), lambda i: (i, 0)),
        pl.BlockSpec((1, D), lambda i: (0, 0)),
        pl.BlockSpec((D, D), lambda i: (0, 0)),
    ],
    out_specs=pl.BlockSpec((BN, D), lambda i: (i, 0)),
    out_shape=jax.ShapeDtypeStruct((NR, D), jnp.float32),
)


def _fin_body(agg_ref, dinv_ref, b_ref, out_ref):
    a = agg_ref[0] + agg_ref[1]
    out_ref[...] = a * dinv_ref[...] + b_ref[...]


BNF = 1000


_fin = pl.pallas_call(
    _fin_body,
    grid=(N // BNF,),
    in_specs=[
        pl.BlockSpec((NC, BNF, D), lambda i: (0, i, 0)),
        pl.BlockSpec((BNF, 1), lambda i: (i, 0)),
        pl.BlockSpec((1, D), lambda i: (0, 0)),
    ],
    out_specs=pl.BlockSpec((BNF, D), lambda i: (i, 0)),
    out_shape=jax.ShapeDtypeStruct((N, D), jnp.float32),
)


def kernel(x, edge_index, W1, b1, W2, b2):
    ei = edge_index.astype(jnp.int32)
    src = ei[0].reshape(NW, NSEG, SEGC, B)
    dst = ei[1].reshape(NW, NSEG, SEGC, B)

    deg2 = _deg(dst)
    x_pad = jnp.pad(x, ((0, NR - N), (0, 0)))
    zeros = jnp.zeros((RPT, D), jnp.float32)

    h1p, dinv = _prep(deg2[0].reshape(NP, 1), deg2[1].reshape(NP, 1), x_pad, W1)
    agg1 = _agg(h1p, src, dst, zeros)
    h2p = _mid(agg1, dinv, b1.reshape(1, D), W2)
    agg2 = _agg(h2p, src, dst, zeros)
    return _fin(agg2, dinv, b2.reshape(1, D))
